# baseline (speedup 1.0000x reference)
import functools

import jax
import jax.numpy as jnp
from jax import lax
from jax.experimental import pallas as pl
from jax.experimental.pallas import tpu as pltpu
from jax.experimental.pallas import tpu_sc as plsc

B = 128
V = 32768
L = 16
NC = 2
NS = 16
NW = NC * NS
ROWS_PER_W = B // NW
U = 8
CHUNK = U * L
VBLK = 2048


def _tc_max_body(x_ref, o_ref):
    i = pl.program_id(0)

    @pl.when(i == 0)
    def _():
        o_ref[...] = jnp.full_like(o_ref, -jnp.inf)

    o_ref[...] = jnp.maximum(o_ref[...], jnp.max(x_ref[...], axis=1)[None, :])


_tc_row_max = pl.pallas_call(
    _tc_max_body,
    grid=(V // VBLK,),
    in_specs=[pl.BlockSpec((B, VBLK), lambda i: (0, i))],
    out_specs=pl.BlockSpec((1, B), lambda i: (0, 0)),
    out_shape=jax.ShapeDtypeStruct((1, B), jnp.float32),
)


def _butterfly(v, op):
    lane = lax.iota(jnp.int32, L)
    for k in (8, 4, 2, 1):
        v = op(v, v.at[lane ^ k].get(mode="promise_in_bounds"))
    return v


def _row_sumexp(buf, base, row_max):
    init = tuple(jnp.zeros((L,), jnp.float32) for _ in range(U))

    @plsc.parallel_loop(0, V, CHUNK, carry=init)
    def ss(off, ss):
        return tuple(
            ss[u] + jnp.exp(buf[pl.ds(base + off + u * L, L)] - row_max)
            for u in range(U)
        )

    s = ss[0]
    for u in range(1, U):
        s = s + ss[u]
    return _butterfly(s, jnp.add)


@functools.partial(
    pl.kernel,
    mesh=plsc.VectorSubcoreMesh(core_axis_name="c", subcore_axis_name="s"),
    out_type=jax.ShapeDtypeStruct((NW, L), jnp.float32),
    scratch_types=[
        pltpu.VMEM((2 * V,), jnp.float32),
        pltpu.VMEM((B,), jnp.float32),
        pltpu.VMEM((L,), jnp.float32),
        pltpu.SemaphoreType.DMA((2,)),
    ],
)
def _sc_top1(x_hbm, mx_hbm, out_hbm, buf, mx_v, out_buf, sems):
    cid = lax.axis_index("c")
    sid = lax.axis_index("s")
    wid = sid * NC + cid
    base_row = wid * ROWS_PER_W

    pltpu.make_async_copy(
        x_hbm.at[base_row], buf.at[pl.ds(0, V)], sems.at[0]
    ).start()
    pltpu.sync_copy(mx_hbm.at[0], mx_v)

    lane = lax.iota(jnp.int32, L)
    mx_chunk = mx_v[pl.ds((wid // (L // ROWS_PER_W)) * L, L)]

    def row_body(r, acc):
        cur = lax.rem(r, 2)
        nxt = lax.rem(r + 1, 2)

        @pl.when(r + 1 < ROWS_PER_W)
        def _():
            pltpu.make_async_copy(
                x_hbm.at[base_row + r + 1],
                buf.at[pl.ds(nxt * V, V)],
                sems.at[nxt],
            ).start()

        pltpu.make_async_copy(
            x_hbm.at[base_row + r], buf.at[pl.ds(cur * V, V)], sems.at[cur]
        ).wait()

        sel = (wid % (L // ROWS_PER_W)) * ROWS_PER_W + r
        row_max = _butterfly(
            jnp.where(lane == sel, mx_chunk, -jnp.inf), jnp.maximum
        )
        sum_exp = _row_sumexp(buf, cur * V, row_max)
        return jnp.where(lane == r, 1.0 / sum_exp, acc)

    acc = lax.fori_loop(0, ROWS_PER_W, row_body, jnp.zeros((L,), jnp.float32))

    out_buf[...] = acc
    pltpu.sync_copy(out_buf, out_hbm.at[wid])


def kernel(class_t, dom_res):
    maxes = _tc_row_max(class_t)
    padded = _sc_top1(class_t, maxes)
    return padded[:, :ROWS_PER_W].reshape(-1)


# speedup vs baseline: 1.0629x; 1.0629x over previous
import functools

import jax
import jax.numpy as jnp
from jax import lax
from jax.experimental import pallas as pl
from jax.experimental.pallas import tpu as pltpu
from jax.experimental.pallas import tpu_sc as plsc

B = 128
V = 32768
L = 16
NC = 2
NS = 16
NW = NC * NS
ROWS_PER_W = B // NW
U = 8
CHUNK = U * L
VBLK = 4096


def _tc_max_body(x_ref, o_ref, acc_ref):
    i = pl.program_id(0)
    m = jnp.max(x_ref[...].reshape(B, VBLK // 128, 128), axis=1)

    @pl.when(i == 0)
    def _():
        acc_ref[...] = m

    @pl.when(i > 0)
    def _():
        acc_ref[...] = jnp.maximum(acc_ref[...], m)

    @pl.when(i == V // VBLK - 1)
    def _():
        o_ref[...] = jnp.max(acc_ref[...], axis=1)[None, :]


_tc_row_max = pl.pallas_call(
    _tc_max_body,
    grid=(V // VBLK,),
    in_specs=[pl.BlockSpec((B, VBLK), lambda i: (0, i))],
    out_specs=pl.BlockSpec((1, B), lambda i: (0, 0)),
    out_shape=jax.ShapeDtypeStruct((1, B), jnp.float32),
    scratch_shapes=[pltpu.VMEM((B, 128), jnp.float32)],
)


def _butterfly(v, op):
    lane = lax.iota(jnp.int32, L)
    for k in (8, 4, 2, 1):
        v = op(v, v.at[lane ^ k].get(mode="promise_in_bounds"))
    return v


def _sumexp_accs(buf, base, n, row_max, init):
    @plsc.parallel_loop(0, n, CHUNK, unroll=2, carry=init)
    def ss(off, ss):
        return tuple(
            ss[u] + jnp.exp(buf[pl.ds(base + off + u * L, L)] - row_max)
            for u in range(U)
        )

    return ss


def _accs_total(ss):
    s = ss[0]
    for u in range(1, U):
        s = s + ss[u]
    return _butterfly(s, jnp.add)


def _zero_accs():
    return tuple(jnp.zeros((L,), jnp.float32) for _ in range(U))


@functools.partial(
    pl.kernel,
    mesh=plsc.VectorSubcoreMesh(core_axis_name="c", subcore_axis_name="s"),
    out_type=jax.ShapeDtypeStruct((NW, L), jnp.float32),
    scratch_types=[
        pltpu.VMEM((2 * V,), jnp.float32),
        pltpu.VMEM((B,), jnp.float32),
        pltpu.VMEM((L,), jnp.float32),
        pltpu.SemaphoreType.DMA((4,)),
    ],
)
def _sc_top1(x_hbm, mx_hbm, out_hbm, buf, mx_v, out_buf, sems):
    cid = lax.axis_index("c")
    sid = lax.axis_index("s")
    wid = sid * NC + cid
    base_row = wid * ROWS_PER_W
    H = V // 2

    pltpu.make_async_copy(
        x_hbm.at[base_row, pl.ds(0, H)], buf.at[pl.ds(0, H)], sems.at[2]
    ).start()
    pltpu.make_async_copy(
        x_hbm.at[base_row, pl.ds(H, H)], buf.at[pl.ds(H, H)], sems.at[3]
    ).start()
    pltpu.make_async_copy(
        x_hbm.at[base_row + 1], buf.at[pl.ds(V, V)], sems.at[1]
    ).start()
    pltpu.sync_copy(mx_hbm.at[0], mx_v)

    lane = lax.iota(jnp.int32, L)
    mx_chunk = mx_v[pl.ds((wid // (L // ROWS_PER_W)) * L, L)]
    sel0 = (wid % (L // ROWS_PER_W)) * ROWS_PER_W

    def bcast_max(sel):
        return _butterfly(
            jnp.where(lane == sel, mx_chunk, -jnp.inf), jnp.maximum
        )

    rm0 = bcast_max(sel0)
    pltpu.make_async_copy(
        x_hbm.at[base_row, pl.ds(0, H)], buf.at[pl.ds(0, H)], sems.at[2]
    ).wait()
    accs = _sumexp_accs(buf, 0, H, rm0, _zero_accs())
    pltpu.make_async_copy(
        x_hbm.at[base_row, pl.ds(H, H)], buf.at[pl.ds(H, H)], sems.at[3]
    ).wait()
    accs = _sumexp_accs(buf, H, H, rm0, accs)
    acc = jnp.where(lane == 0, 1.0 / _accs_total(accs), jnp.zeros((L,), jnp.float32))

    def row_body(r, acc):
        cur = lax.rem(r, 2)
        nxt = lax.rem(r + 1, 2)

        @pl.when(r + 1 < ROWS_PER_W)
        def _():
            pltpu.make_async_copy(
                x_hbm.at[base_row + r + 1],
                buf.at[pl.ds(nxt * V, V)],
                sems.at[nxt],
            ).start()

        pltpu.make_async_copy(
            x_hbm.at[base_row + r], buf.at[pl.ds(cur * V, V)], sems.at[cur]
        ).wait()

        row_max = bcast_max(sel0 + r)
        sum_exp = _accs_total(_sumexp_accs(buf, cur * V, V, row_max, _zero_accs()))
        return jnp.where(lane == r, 1.0 / sum_exp, acc)

    acc = lax.fori_loop(1, ROWS_PER_W, row_body, acc)

    out_buf[...] = acc
    pltpu.sync_copy(out_buf, out_hbm.at[wid])


def kernel(class_t, dom_res):
    maxes = _tc_row_max(class_t)
    padded = _sc_top1(class_t, maxes)
    return padded[:, :ROWS_PER_W].reshape(-1)
